# baseline (speedup 1.0000x reference)
import functools

import jax
import jax.numpy as jnp
from jax import lax
from jax.experimental import pallas as pl
from jax.experimental.pallas import tpu as pltpu
from jax.experimental.pallas import tpu_sc as plsc

_INFO = plsc.get_sparse_core_info()
_NC, _NS = _INFO.num_cores, _INFO.num_subcores
_NW = _NC * _NS


def _make_sc_copy(seq_len, dim, dtype):
    rows_per_w = seq_len // _NW
    mesh = plsc.VectorSubcoreMesh(core_axis_name="c", subcore_axis_name="s")

    @functools.partial(
        pl.kernel,
        mesh=mesh,
        out_type=jax.ShapeDtypeStruct((seq_len, dim), dtype),
        scratch_types=[pltpu.SemaphoreType.DMA],
    )
    def sc_copy(w_hbm, out_hbm, sem):
        wid = lax.axis_index("s") * _NC + lax.axis_index("c")
        base = wid * rows_per_w
        pltpu.async_copy(
            w_hbm.at[pl.ds(base, rows_per_w)],
            out_hbm.at[pl.ds(base, rows_per_w)],
            sem,
        ).wait()

    return sc_copy


def kernel(inputs, weight):
    bsz, seq_len = inputs.shape[:2]
    dim = weight.shape[1]
    return _make_sc_copy(seq_len, dim, weight.dtype)(weight)


# speedup vs baseline: 24.3708x; 24.3708x over previous
import functools

import jax
import jax.numpy as jnp
from jax import lax
from jax.experimental import pallas as pl
from jax.experimental.pallas import tpu as pltpu
from jax.experimental.pallas import tpu_sc as plsc

_INFO = plsc.get_sparse_core_info()
_NC, _NS = _INFO.num_cores, _INFO.num_subcores
_NW = _NC * _NS
_CHUNK_ROWS = 32


def _make_sc_copy(seq_len, dim, dtype):
    rows_per_w = seq_len // _NW
    n_chunks = rows_per_w // _CHUNK_ROWS
    mesh = plsc.VectorSubcoreMesh(core_axis_name="c", subcore_axis_name="s")

    @functools.partial(
        pl.kernel,
        mesh=mesh,
        out_type=jax.ShapeDtypeStruct((seq_len, dim), dtype),
        scratch_types=[
            pltpu.VMEM((2, _CHUNK_ROWS, dim), dtype),
            pltpu.SemaphoreType.DMA((2,)),
            pltpu.SemaphoreType.DMA((2,)),
        ],
    )
    def sc_copy(w_hbm, out_hbm, buf, rsem, wsem):
        wid = lax.axis_index("s") * _NC + lax.axis_index("c")
        base = wid * rows_per_w

        def rd(i):
            return pltpu.make_async_copy(
                w_hbm.at[pl.ds(base + i * _CHUNK_ROWS, _CHUNK_ROWS)],
                buf.at[i % 2],
                rsem.at[i % 2],
            )

        def wr(i):
            return pltpu.make_async_copy(
                buf.at[i % 2],
                out_hbm.at[pl.ds(base + i * _CHUNK_ROWS, _CHUNK_ROWS)],
                wsem.at[i % 2],
            )

        rd(0).start()
        for i in range(n_chunks):
            if i + 1 < n_chunks:
                if i >= 1:
                    wr(i - 1).wait()
                rd(i + 1).start()
            rd(i).wait()
            wr(i).start()
        wr(n_chunks - 1).wait()
        if n_chunks >= 2:
            wr(n_chunks - 2).wait()

    return sc_copy


def kernel(inputs, weight):
    bsz, seq_len = inputs.shape[:2]
    dim = weight.shape[1]
    return _make_sc_copy(seq_len, dim, weight.dtype)(weight)
